# Initial kernel scaffold; baseline (speedup 1.0000x reference)
import functools

import jax
import jax.numpy as jnp
from jax import lax
from jax.experimental import pallas as pl
from jax.experimental.pallas import tpu as pltpu
from jax.experimental.pallas import tpu_sc as plsc

_D = 64
_B_TOTAL = 16384 * 50
_NC = 2
_NS = 16
_NW = _NC * _NS
_B_PER_W = _B_TOTAL // _NW
_IDXW = 128
_K = 8
_CHUNK = _IDXW * _K
_NCHUNK = _B_PER_W // _CHUNK


@functools.partial(
    pl.kernel,
    out_type=jax.ShapeDtypeStruct((_B_TOTAL, _D), jnp.float32),
    mesh=plsc.VectorSubcoreMesh(core_axis_name="c", subcore_axis_name="s"),
    scratch_types=[
        pltpu.VMEM((_K, _IDXW), jnp.int32),
        pltpu.VMEM((_CHUNK, _D), jnp.float32),
        pltpu.SemaphoreType.DMA,
    ],
)
def _gather_kernel(idx_hbm, table_hbm, out_hbm, idx_v, rows_v, sem):
    wid = lax.axis_index("s") * _NC + lax.axis_index("c")
    row_base = wid * _B_PER_W

    def body(i, carry):
        off = row_base + i * _CHUNK
        pltpu.sync_copy(idx_hbm.at[pl.ds(off // _IDXW, _K)], idx_v)
        copies = []
        for j in range(_K):
            copies.append(
                pltpu.async_copy(
                    table_hbm.at[idx_v.at[j]],
                    rows_v.at[pl.ds(j * _IDXW, _IDXW)],
                    sem,
                )
            )
        for c in copies:
            c.wait()
        pltpu.sync_copy(rows_v, out_hbm.at[pl.ds(off, _CHUNK)])
        return carry

    lax.fori_loop(0, _NCHUNK, body, 0)


def kernel(x, embeddings):
    idx = x.reshape(_B_TOTAL // _IDXW, _IDXW).astype(jnp.int32)
    out = _gather_kernel(idx, embeddings)
    return out.reshape(x.shape[0], x.shape[1], _D)


# speedup vs baseline: 1.8430x; 1.8430x over previous
import functools

import jax
import jax.numpy as jnp
from jax import lax
from jax.experimental import pallas as pl
from jax.experimental.pallas import tpu as pltpu
from jax.experimental.pallas import tpu_sc as plsc

_D = 64
_B_TOTAL = 16384 * 50
_NC = 2
_NS = 16
_NW = _NC * _NS
_B_PER_W = _B_TOTAL // _NW
_IDXW = 128
_K = 8
_CHUNK = _IDXW * _K
_NCHUNK = _B_PER_W // _CHUNK


@functools.partial(
    pl.kernel,
    out_type=jax.ShapeDtypeStruct((_B_TOTAL, _D), jnp.float32),
    mesh=plsc.VectorSubcoreMesh(core_axis_name="c", subcore_axis_name="s"),
    scratch_types=[
        pltpu.VMEM((_K, _IDXW), jnp.int32),
        pltpu.VMEM((_CHUNK, _D), jnp.float32),
        pltpu.SemaphoreType.DMA,
    ],
    compiler_params=pltpu.CompilerParams(use_tc_tiling_on_sc=False),
)
def _gather_kernel(idx_hbm, table_hbm, out_hbm, idx_v, rows_v, sem):
    wid = lax.axis_index("s") * _NC + lax.axis_index("c")
    row_base = wid * _B_PER_W

    def body(i, carry):
        off = pl.multiple_of(row_base + i * _CHUNK, _CHUNK)
        pltpu.sync_copy(idx_hbm.at[pl.ds(pl.multiple_of(off // _IDXW, _K), _K)], idx_v)
        copies = []
        for j in range(_K):
            copies.append(
                pltpu.async_copy(
                    table_hbm.at[idx_v.at[j]],
                    rows_v.at[pl.ds(j * _IDXW, _IDXW)],
                    sem,
                )
            )
        for c in copies:
            c.wait()
        pltpu.sync_copy(rows_v, out_hbm.at[pl.ds(off, _CHUNK)])
        return carry

    lax.fori_loop(0, _NCHUNK, body, 0)


def kernel(x, embeddings):
    idx = x.reshape(_B_TOTAL // _IDXW, _IDXW).astype(jnp.int32)
    out = _gather_kernel(idx, embeddings)
    return out.reshape(x.shape[0], x.shape[1], _D)


# speedup vs baseline: 1.8631x; 1.0109x over previous
import functools

import jax
import jax.numpy as jnp
from jax import lax
from jax.experimental import pallas as pl
from jax.experimental.pallas import tpu as pltpu
from jax.experimental.pallas import tpu_sc as plsc

_D = 64
_B_TOTAL = 16384 * 50
_NC = 2
_NS = 16
_NW = _NC * _NS
_B_PER_W = _B_TOTAL // _NW
_IDXW = 128
_K = 4
_R = _IDXW * _K
_NCHUNK = _B_PER_W // _R
_IDX_ROWS = _B_PER_W // _IDXW


@functools.partial(
    pl.kernel,
    out_type=jax.ShapeDtypeStruct((_B_TOTAL, _D), jnp.float32),
    mesh=plsc.VectorSubcoreMesh(core_axis_name="c", subcore_axis_name="s"),
    scratch_types=[
        pltpu.VMEM((_IDX_ROWS, _IDXW), jnp.int32),
        pltpu.VMEM((2 * _R, _D), jnp.float32),
        pltpu.SemaphoreType.DMA,
        pltpu.SemaphoreType.DMA,
        pltpu.SemaphoreType.DMA,
        pltpu.SemaphoreType.DMA,
    ],
    compiler_params=pltpu.CompilerParams(use_tc_tiling_on_sc=False),
)
def _gather_kernel(idx_hbm, table_hbm, out_hbm, idx_v, rows_v, g0, g1, o0, o1):
    wid = lax.axis_index("s") * _NC + lax.axis_index("c")
    row_base = pl.multiple_of(wid * _B_PER_W, _B_PER_W)

    pltpu.sync_copy(
        idx_hbm.at[pl.ds(pl.multiple_of(wid * _IDX_ROWS, _IDX_ROWS), _IDX_ROWS)],
        idx_v,
    )

    gsems = (g0, g1)
    osems = (o0, o1)

    def fire_gathers(g, b, gsem):
        for j in range(_K):
            pltpu.async_copy(
                table_hbm.at[idx_v.at[g * _K + j]],
                rows_v.at[pl.ds(b * _R + j * _IDXW, _IDXW)],
                gsem,
            )

    def wait_gathers(b, gsem):
        pltpu.make_async_copy(
            out_hbm.at[pl.ds(0, _R)],
            rows_v.at[pl.ds(b * _R, _R)],
            gsem,
        ).wait()

    def fire_store(g, b, osem):
        off = pl.multiple_of(row_base + g * _R, _R)
        pltpu.async_copy(
            rows_v.at[pl.ds(b * _R, _R)],
            out_hbm.at[pl.ds(off, _R)],
            osem,
        )

    def wait_store(b, osem):
        pltpu.make_async_copy(
            rows_v.at[pl.ds(b * _R, _R)],
            out_hbm.at[pl.ds(0, _R)],
            osem,
        ).wait()

    def body(i, carry):
        for b in range(2):
            g = i * 2 + b
            @pl.when(i >= 1)
            def _():
                wait_store(b, osems[b])
            fire_gathers(g, b, gsems[b])
        for b in range(2):
            g = i * 2 + b
            wait_gathers(b, gsems[b])
            fire_store(g, b, osems[b])
        return carry

    lax.fori_loop(0, _NCHUNK // 2, body, 0)
    wait_store(0, o0)
    wait_store(1, o1)


def kernel(x, embeddings):
    idx = x.reshape(_B_TOTAL // _IDXW, _IDXW).astype(jnp.int32)
    out = _gather_kernel(idx, embeddings)
    return out.reshape(x.shape[0], x.shape[1], _D)


# speedup vs baseline: 1.8660x; 1.0016x over previous
import functools

import jax
import jax.numpy as jnp
from jax import lax
from jax.experimental import pallas as pl
from jax.experimental.pallas import tpu as pltpu
from jax.experimental.pallas import tpu_sc as plsc

_D = 64
_B_TOTAL = 16384 * 50
_NC = 2
_NS = 16
_NW = _NC * _NS
_B_PER_W = _B_TOTAL // _NW
_IDXW = 128
_K = 4
_R = _IDXW * _K
_NCHUNK = _B_PER_W // _R


@functools.partial(
    pl.kernel,
    out_type=jax.ShapeDtypeStruct((_B_TOTAL, _D), jnp.float32),
    mesh=plsc.VectorSubcoreMesh(core_axis_name="c", subcore_axis_name="s"),
    scratch_types=[
        pltpu.VMEM((_B_PER_W,), jnp.int32),
        pltpu.VMEM((2 * _R, _D), jnp.float32),
        pltpu.SemaphoreType.DMA,
        pltpu.SemaphoreType.DMA,
        pltpu.SemaphoreType.DMA,
        pltpu.SemaphoreType.DMA,
    ],
    compiler_params=pltpu.CompilerParams(use_tc_tiling_on_sc=False),
)
def _gather_kernel(idx_hbm, table_hbm, out_hbm, idx_v, rows_v, g0, g1, o0, o1):
    wid = lax.axis_index("s") * _NC + lax.axis_index("c")
    row_base = pl.multiple_of(wid * _B_PER_W, _B_PER_W)

    pltpu.sync_copy(idx_hbm.at[pl.ds(row_base, _B_PER_W)], idx_v)

    gsems = (g0, g1)
    osems = (o0, o1)

    def fire_gathers(g, b, gsem):
        base = pl.multiple_of(g * _R, _R)
        for j in range(_K):
            pltpu.async_copy(
                table_hbm.at[idx_v.at[pl.ds(base + j * _IDXW, _IDXW)]],
                rows_v.at[pl.ds(b * _R + j * _IDXW, _IDXW)],
                gsem,
            )

    def wait_gathers(b, gsem):
        pltpu.make_async_copy(
            out_hbm.at[pl.ds(0, _R)],
            rows_v.at[pl.ds(b * _R, _R)],
            gsem,
        ).wait()

    def fire_store(g, b, osem):
        off = pl.multiple_of(row_base + g * _R, _R)
        pltpu.async_copy(
            rows_v.at[pl.ds(b * _R, _R)],
            out_hbm.at[pl.ds(off, _R)],
            osem,
        )

    def wait_store(b, osem):
        pltpu.make_async_copy(
            rows_v.at[pl.ds(b * _R, _R)],
            out_hbm.at[pl.ds(0, _R)],
            osem,
        ).wait()

    def body(i, carry):
        for b in range(2):
            g = i * 2 + b
            @pl.when(i >= 1)
            def _():
                wait_store(b, osems[b])
            fire_gathers(g, b, gsems[b])
        for b in range(2):
            g = i * 2 + b
            wait_gathers(b, gsems[b])
            fire_store(g, b, osems[b])
        return carry

    lax.fori_loop(0, _NCHUNK // 2, body, 0)
    wait_store(0, o0)
    wait_store(1, o1)


def kernel(x, embeddings):
    idx = x.reshape(_B_TOTAL).astype(jnp.int32)
    out = _gather_kernel(idx, embeddings)
    return out.reshape(x.shape[0], x.shape[1], _D)
